# baseline (speedup 1.0000x reference)
import jax
import jax.numpy as jnp
from jax.experimental import pallas as pl
from jax.experimental.pallas import tpu as pltpu

_C = 8
_BN = 262144
_CN = 16384
_NCORES = 2


def _onehot_bf16(x):
    m = jnp.max(x, axis=0, keepdims=True)
    return jnp.where(x == m, 1.0, 0.0).astype(jnp.bfloat16)


def _cm_kernel(xt_ref, xp_ref, out_ref):
    j = pl.program_id(1)

    @pl.when(j == 0)
    def _():
        out_ref[...] = jnp.zeros_like(out_ref)

    acc = jnp.zeros((_C, _C), jnp.float32)
    for c in range(_BN // _CN):
        sl = slice(c * _CN, (c + 1) * _CN)
        oh_t = _onehot_bf16(xt_ref[:, sl])
        oh_p = _onehot_bf16(xp_ref[:, sl])
        acc = acc + jax.lax.dot_general(
            oh_t, oh_p, (((1,), (1,)), ((), ())),
            preferred_element_type=jnp.float32,
        )
    out_ref[...] += acc


def kernel(y_true, y_pred):
    n = y_true.shape[0]
    xt = y_true.T
    xp = y_pred.T
    k = n // (_BN * _NCORES)
    g = pl.pallas_call(
        _cm_kernel,
        grid=(_NCORES, k),
        in_specs=[
            pl.BlockSpec((_C, _BN), lambda i, j: (0, i * k + j)),
            pl.BlockSpec((_C, _BN), lambda i, j: (0, i * k + j)),
        ],
        out_specs=pl.BlockSpec((None, _C, _C), lambda i, j: (i, 0, 0)),
        out_shape=jax.ShapeDtypeStruct((_NCORES, _C, _C), jnp.float32),
        compiler_params=pltpu.CompilerParams(
            dimension_semantics=("parallel", "arbitrary"),
        ),
    )(xt, xp)
    return g.sum(axis=0)


# speedup vs baseline: 1.0173x; 1.0173x over previous
import jax
import jax.numpy as jnp
from jax.experimental import pallas as pl
from jax.experimental.pallas import tpu as pltpu

_C = 8
_BN = 262144
_CN = 8192
_NCORES = 2


def _onehot_bf16(x):
    m = jnp.max(x, axis=0, keepdims=True)
    return jnp.where(x == m, 1.0, 0.0).astype(jnp.bfloat16)


def _cm_kernel(xt_ref, xp_ref, out_ref):
    j = pl.program_id(1)

    @pl.when(j == 0)
    def _():
        out_ref[...] = jnp.zeros_like(out_ref)

    acc = jnp.zeros((_C, _C), jnp.float32)
    for c in range(_BN // _CN):
        sl = slice(c * _CN, (c + 1) * _CN)
        oh_t = _onehot_bf16(xt_ref[:, sl])
        oh_p = _onehot_bf16(xp_ref[:, sl])
        acc = acc + jax.lax.dot_general(
            oh_t, oh_p, (((1,), (1,)), ((), ())),
            preferred_element_type=jnp.float32,
        )
    out_ref[...] += acc


def kernel(y_true, y_pred):
    n = y_true.shape[0]
    xt = y_true.T
    xp = y_pred.T
    k = n // (_BN * _NCORES)
    g = pl.pallas_call(
        _cm_kernel,
        grid=(_NCORES, k),
        in_specs=[
            pl.BlockSpec((_C, _BN), lambda i, j: (0, i * k + j)),
            pl.BlockSpec((_C, _BN), lambda i, j: (0, i * k + j)),
        ],
        out_specs=pl.BlockSpec((None, _C, _C), lambda i, j: (i, 0, 0)),
        out_shape=jax.ShapeDtypeStruct((_NCORES, _C, _C), jnp.float32),
        compiler_params=pltpu.CompilerParams(
            dimension_semantics=("parallel", "arbitrary"),
        ),
    )(xt, xp)
    return g.sum(axis=0)
